# baseline (speedup 1.0000x reference)
import functools

import jax
import jax.numpy as jnp
import numpy as np
from jax import lax
from jax.experimental import pallas as pl
from jax.experimental.pallas import tpu as pltpu
from jax.experimental.pallas import tpu_sc as plsc

N = 10000
NPAD = 10240
IN = 128
H = 8
C = 16
HID = H * C
OUT = 64

NC = 2
NS = 16
NW = NC * NS
CH = 96
RC = 64
BL = 512

_R128 = np.zeros((16, HID), np.float32)
for _h in range(H):
    _R128[_h, _h * C:(_h + 1) * C] = 1.0
_R64 = np.zeros((16, OUT), np.float32)
_R64[0, :] = 1.0
_KRON = np.kron(np.eye(H, dtype=np.float32), np.ones((C, 1), np.float32))


def _ln(t, g, b):
    m = jnp.mean(t, axis=-1, keepdims=True)
    v = jnp.mean((t - m) * (t - m), axis=-1, keepdims=True)
    return (t - m) * lax.rsqrt(v + 1e-5) * g + b


def _elu(t):
    return jnp.where(t > 0, t, jnp.exp(jnp.minimum(t, 0.0)) - 1.0)


def _pre_body(x_ref, win_ref, bin_ref, g_ref, b_ref, w0_ref, as_ref, ad_ref,
              h_ref, hw_ref, ast_ref, adt_ref):
    t = jnp.dot(x_ref[...], win_ref[...],
                preferred_element_type=jnp.float32) + bin_ref[...]
    h = _elu(_ln(t, g_ref[...], b_ref[...]))
    h_ref[...] = h
    hw = jnp.dot(h, w0_ref[...], preferred_element_type=jnp.float32)
    hw_ref[...] = hw
    ast_ref[...] = jnp.dot(hw, as_ref[...], preferred_element_type=jnp.float32)
    adt_ref[...] = jnp.dot(hw, ad_ref[...], preferred_element_type=jnp.float32)


def _mid_body(acc_ref, den_ref, bias_ref, g_ref, b_ref, r_ref, hprev_ref,
              wn_ref, as_ref, ad_ref, res_ref, hw_ref, ast_ref, adt_ref):
    a = acc_ref[0] + acc_ref[1]
    d = den_ref[0] + den_ref[1]
    db = jnp.dot(d, r_ref[...], preferred_element_type=jnp.float32) + 1e-16
    o = a / db + bias_ref[...]
    o = _elu(_ln(o, g_ref[...], b_ref[...])) + hprev_ref[...]
    res_ref[...] = o
    hw = jnp.dot(o, wn_ref[...], preferred_element_type=jnp.float32)
    hw_ref[...] = hw
    ast_ref[...] = jnp.dot(hw, as_ref[...], preferred_element_type=jnp.float32)
    adt_ref[...] = jnp.dot(hw, ad_ref[...], preferred_element_type=jnp.float32)


def _fin_body(acc_ref, den_ref, bias_ref, g_ref, b_ref, r_ref, out_ref):
    a = acc_ref[0] + acc_ref[1]
    d = den_ref[0] + den_ref[1]
    db = jnp.dot(d, r_ref[...], preferred_element_type=jnp.float32) + 1e-16
    out_ref[...] = _ln(a / db + bias_ref[...], g_ref[...], b_ref[...])


def _row_spec(width):
    return pl.BlockSpec((BL, width), lambda i: (i, 0))


def _full_spec(shape):
    return pl.BlockSpec(shape, lambda i: tuple(0 for _ in shape))


def _pre_call(x, w_in, b_in, g, b, w0, a0s, a0d):
    grid = (NPAD // BL,)
    return pl.pallas_call(
        _pre_body,
        grid=grid,
        in_specs=[
            _row_spec(IN),
            _full_spec((IN, HID)),
            _full_spec((1, HID)),
            _full_spec((1, HID)),
            _full_spec((1, HID)),
            _full_spec((HID, HID)),
            _full_spec((HID, 16)),
            _full_spec((HID, 16)),
        ],
        out_specs=[_row_spec(HID), _row_spec(HID), _row_spec(16),
                   _row_spec(16)],
        out_shape=[
            jax.ShapeDtypeStruct((NPAD, HID), jnp.float32),
            jax.ShapeDtypeStruct((NPAD, HID), jnp.float32),
            jax.ShapeDtypeStruct((NPAD, 16), jnp.float32),
            jax.ShapeDtypeStruct((NPAD, 16), jnp.float32),
        ],
    )(x, w_in, b_in, g, b, w0, a0s, a0d)


def _mid_call(acc, den, bias, g, b, rmat, hprev, wn, ans, and_, hid_next):
    grid = (NPAD // BL,)
    return pl.pallas_call(
        _mid_body,
        grid=grid,
        in_specs=[
            pl.BlockSpec((2, BL, HID), lambda i: (0, i, 0)),
            pl.BlockSpec((2, BL, 16), lambda i: (0, i, 0)),
            _full_spec((1, HID)),
            _full_spec((1, HID)),
            _full_spec((1, HID)),
            _full_spec((16, HID)),
            _row_spec(HID),
            _full_spec((HID, hid_next)),
            _full_spec((hid_next, 16)),
            _full_spec((hid_next, 16)),
        ],
        out_specs=[_row_spec(HID), _row_spec(hid_next), _row_spec(16),
                   _row_spec(16)],
        out_shape=[
            jax.ShapeDtypeStruct((NPAD, HID), jnp.float32),
            jax.ShapeDtypeStruct((NPAD, hid_next), jnp.float32),
            jax.ShapeDtypeStruct((NPAD, 16), jnp.float32),
            jax.ShapeDtypeStruct((NPAD, 16), jnp.float32),
        ],
    )(acc, den, bias, g, b, rmat, hprev, wn, ans, and_)


def _fin_call(acc, den, bias, g, b, rmat):
    grid = (NPAD // BL,)
    return pl.pallas_call(
        _fin_body,
        grid=grid,
        in_specs=[
            pl.BlockSpec((2, BL, OUT), lambda i: (0, i, 0)),
            pl.BlockSpec((2, BL, 16), lambda i: (0, i, 0)),
            _full_spec((1, OUT)),
            _full_spec((1, OUT)),
            _full_spec((1, OUT)),
            _full_spec((16, OUT)),
        ],
        out_specs=[_row_spec(OUT)],
        out_shape=[jax.ShapeDtypeStruct((NPAD, OUT), jnp.float32)],
    )(acc, den, bias, g, b, rmat)[0]


def _make_edge_pass(hid, ew):
    nreg = hid // 16
    nch = ew // CH
    assert nch % 4 == 0 and nch >= 4
    t_max = nch // 4
    rows_per = NPAD // NS
    nzc = rows_per // RC
    heads8 = hid == HID
    mesh = plsc.VectorSubcoreMesh(core_axis_name="c", subcore_axis_name="s")

    @functools.partial(
        pl.kernel,
        out_type=[
            jax.ShapeDtypeStruct((NC, NPAD, hid), jnp.float32),
            jax.ShapeDtypeStruct((NC, NPAD, 16), jnp.float32),
        ],
        mesh=mesh,
        scratch_types=(
            [pltpu.VMEM((2, CH), jnp.int32)] * 4 +
            [pltpu.VMEM((CH, hid), jnp.float32)] * 2 +
            [pltpu.VMEM((CH, 16), jnp.float32)] * 6 +
            [
                pltpu.VMEM_SHARED((NPAD, hid), jnp.float32),
                pltpu.VMEM_SHARED((NPAD, 16), jnp.float32),
            ] +
            [pltpu.SemaphoreType.DMA] * 5
        ),
        compiler_params=pltpu.CompilerParams(use_tc_tiling_on_sc=False),
    )
    def edge_pass(sd_hbm, hw_hbm, as_hbm, ad_hbm, acc_out, den_out,
                  sd0, sd1, sd2, sd3, hg_a, hg_b,
                  asg_a, asg_b, adg_a, adg_b, exu_a, exu_b,
                  acc_s, den_s, semi, semg_a, semg_b, sems_a, sems_b):
        cid = lax.axis_index("c")
        sid = lax.axis_index("s")
        wid = sid * NC + cid

        sd4 = [sd0, sd1, sd2, sd3]
        sv4 = [r.at[0] for r in sd4]
        dv4 = [r.at[1] for r in sd4]
        hg2 = [hg_a, hg_b]
        asg2 = [asg_a, asg_b]
        adg2 = [adg_a, adg_b]
        exu2 = [exu_a, exu_b]
        semg2 = [semg_a, semg_b]
        sems2 = [sems_a, sems_b]

        zero16 = jnp.zeros((16,), jnp.float32)

        def zero_row(r, _):
            for j in range(nreg):
                hg_a[r, pl.ds(j * 16, 16)] = zero16
            exu_a[r, :] = zero16
            return 0

        lax.fori_loop(0, CH, zero_row, 0)

        base_r = sid * rows_per

        def zero_spmem(k, _):
            pltpu.async_copy(hg_a.at[pl.ds(0, RC)],
                             acc_s.at[pl.ds(base_r + k * RC, RC)], semi)
            pltpu.async_copy(exu_a.at[pl.ds(0, RC)],
                             den_s.at[pl.ds(base_r + k * RC, RC)], semi)
            return 0

        def zero_wait(k, _):
            pltpu.make_async_copy(
                hg_a.at[pl.ds(0, RC)], acc_s.at[pl.ds(base_r, RC)],
                semi).wait()
            pltpu.make_async_copy(
                exu_a.at[pl.ds(0, RC)], den_s.at[pl.ds(base_r, RC)],
                semi).wait()
            return 0

        lax.fori_loop(0, nzc, zero_spmem, 0)
        lax.fori_loop(0, nzc, zero_wait, 0)
        plsc.subcore_barrier()

        bc_idx = [
            jnp.full((16,), j if heads8 else 0, jnp.int32)
            for j in range(nreg)
        ]

        base_c = wid * (ew // CH)

        def issue_idx(ci, sl):
            pltpu.async_copy(sd_hbm.at[base_c + ci], sd4[sl], semi)

        def wait_idx(sl):
            pltpu.make_async_copy(sd_hbm.at[0], sd4[sl], semi).wait()

        def issue_gath(sl, b):
            pltpu.async_copy(hw_hbm.at[sv4[sl]], hg2[b], semg2[b])
            pltpu.async_copy(as_hbm.at[sv4[sl]], asg2[b], semg2[b])
            pltpu.async_copy(ad_hbm.at[dv4[sl]], adg2[b], semg2[b])

        def wait_gath(sl, b):
            pltpu.make_async_copy(hw_hbm.at[sv4[sl]], hg2[b], semg2[b]).wait()
            pltpu.make_async_copy(
                as_hbm.at[sv4[sl]], asg2[b], semg2[b]).wait()
            pltpu.make_async_copy(
                ad_hbm.at[dv4[sl]], adg2[b], semg2[b]).wait()

        def issue_scat(sl, b):
            pltpu.async_copy(hg2[b], acc_s.at[dv4[sl]], sems2[b], add=True)
            pltpu.async_copy(exu2[b], den_s.at[dv4[sl]], sems2[b], add=True)

        def wait_scat(sl, b):
            pltpu.make_async_copy(
                hg2[b], acc_s.at[dv4[sl]], sems2[b]).wait()
            pltpu.make_async_copy(
                exu2[b], den_s.at[dv4[sl]], sems2[b]).wait()

        def compute(b):
            hg, asg, adg, exu = hg2[b], asg2[b], adg2[b], exu2[b]

            @plsc.parallel_loop(0, CH, 1, unroll=6)
            def edge_body(e):
                x = asg[e, :] + adg[e, :]
                ex = jnp.exp(jnp.maximum(x, 0.2 * x))
                exu[e, :] = ex
                for j in range(nreg):
                    bc = ex.at[bc_idx[j]].get(mode="promise_in_bounds")
                    hg[e, pl.ds(j * 16, 16)] = hg[e, pl.ds(j * 16, 16)] * bc

        issue_idx(jnp.int32(0), 0)
        issue_idx(jnp.int32(1), 1)
        wait_idx(0)
        issue_gath(0, 0)

        def body4(t, _):
            for k in range(4):
                g = 4 * t + k
                b = k % 2
                bn = (k + 1) % 2
                sl = k
                sln = (k + 1) % 4
                slp = (k - 1) % 4
                wait_gath(sl, b)
                if k < 3:
                    wait_idx(sln)
                    if k == 0:
                        @pl.when(t > 0)
                        def _():
                            wait_scat(slp, bn)
                    else:
                        wait_scat(slp, bn)
                    issue_gath(sln, bn)
                else:
                    @pl.when(t < t_max - 1)
                    def _():
                        wait_idx(sln)
                        wait_scat(slp, bn)
                        issue_gath(sln, bn)
                compute(b)
                issue_scat(sl, b)
                if k < 2:
                    issue_idx(g + 2, (k + 2) % 4)
                else:
                    @pl.when(t < t_max - 1)
                    def _():
                        issue_idx(g + 2, (k + 2) % 4)
            return 0

        lax.fori_loop(0, t_max, body4, 0)
        wait_scat(2, 0)
        wait_scat(3, 1)
        plsc.subcore_barrier()

        def copy_out(k, _):
            sl = pl.ds(base_r + k * RC, RC)
            pltpu.async_copy(acc_s.at[sl], acc_out.at[cid, sl], semi)
            pltpu.async_copy(den_s.at[sl], den_out.at[cid, sl], semi)
            return 0

        def copy_wait(k, _):
            sl = pl.ds(base_r + k * RC, RC)
            pltpu.make_async_copy(
                acc_s.at[sl], acc_out.at[cid, sl], semi).wait()
            pltpu.make_async_copy(
                den_s.at[sl], den_out.at[cid, sl], semi).wait()
            return 0

        lax.fori_loop(0, nzc, copy_out, 0)
        lax.fori_loop(0, nzc, copy_wait, 0)

    return edge_pass


def _amats(att_s, att_d, hid_in):
    z8 = jnp.zeros((HID, 8), jnp.float32)
    if hid_in == HID:
        asf = att_s.reshape(HID, 1)
        adf = att_d.reshape(HID, 1)
        amat_s = jnp.concatenate([jnp.asarray(_KRON) * asf, z8], axis=1)
        amat_d = jnp.concatenate([jnp.asarray(_KRON) * adf, z8], axis=1)
        return amat_s, amat_d
    z15 = jnp.zeros((OUT, 15), jnp.float32)
    amat_s = jnp.concatenate([att_s.reshape(OUT, 1), z15], axis=1)
    amat_d = jnp.concatenate([att_d.reshape(OUT, 1), z15], axis=1)
    return amat_s, amat_d


def kernel(x, edge_index, W_in, b_in, ln_in_g, ln_in_b, W0, as0, ad0, bc0,
           ln0_g, ln0_b, W1, as1, ad1, bc1, ln1_g, ln1_b, W2, as2, ad2, bc2,
           ln2_g, ln2_b):
    e0 = edge_index.shape[1]
    etot = e0 + N
    nch = 4 * (-(-etot // (NW * CH * 4)))
    ew = nch * CH
    ep = NW * ew
    npad_extra = NPAD - N

    loops = jnp.arange(N, dtype=jnp.int32)
    padc = ep - etot
    pad_idx = (N + (jnp.arange(padc, dtype=jnp.int32) % npad_extra))
    src_all = jnp.concatenate([edge_index[0].astype(jnp.int32), loops, pad_idx])
    dst_all = jnp.concatenate([edge_index[1].astype(jnp.int32), loops, pad_idx])
    sd_all = jnp.stack([src_all.reshape(NW * nch, CH),
                        dst_all.reshape(NW * nch, CH)], axis=1)

    xp = jnp.concatenate([x, jnp.zeros((npad_extra, IN), jnp.float32)], axis=0)

    r2 = lambda v: v.reshape(1, -1)
    r128 = jnp.asarray(_R128)
    r64 = jnp.asarray(_R64)

    edge128 = _make_edge_pass(HID, ew)
    edge64 = _make_edge_pass(OUT, ew)

    a0s, a0d = _amats(as0, ad0, HID)
    a1s, a1d = _amats(as1, ad1, HID)
    a2s, a2d = _amats(as2, ad2, OUT)

    h0, hw0, ast0, adt0 = _pre_call(xp, W_in, r2(b_in), r2(ln_in_g),
                                    r2(ln_in_b), W0, a0s, a0d)
    acc0, den0 = edge128(sd_all, hw0, ast0, adt0)
    h1, hw1, ast1, adt1 = _mid_call(acc0, den0, r2(bc0), r2(ln0_g),
                                    r2(ln0_b), r128, h0, W1, a1s, a1d, HID)
    acc1, den1 = edge128(sd_all, hw1, ast1, adt1)
    h2, hw2, ast2, adt2 = _mid_call(acc1, den1, r2(bc1), r2(ln1_g),
                                    r2(ln1_b), r128, h1, W2, a2s, a2d, OUT)
    acc2, den2 = edge64(sd_all, hw2, ast2, adt2)
    out = _fin_call(acc2, den2, r2(bc2), r2(ln2_g), r2(ln2_b), r64)
    return out[:N]


# speedup vs baseline: 1.1028x; 1.1028x over previous
import functools

import jax
import jax.numpy as jnp
import numpy as np
from jax import lax
from jax.experimental import pallas as pl
from jax.experimental.pallas import tpu as pltpu
from jax.experimental.pallas import tpu_sc as plsc

N = 10000
NPAD = 10240
IN = 128
H = 8
C = 16
HID = H * C
OUT = 64

NC = 2
NS = 16
NW = NC * NS
CH = 104
RC = 64
BL = 512

_R128 = np.zeros((16, HID), np.float32)
for _h in range(H):
    _R128[_h, _h * C:(_h + 1) * C] = 1.0
_R64 = np.zeros((16, OUT), np.float32)
_R64[0, :] = 1.0
_KRON = np.kron(np.eye(H, dtype=np.float32), np.ones((C, 1), np.float32))


def _ln(t, g, b):
    m = jnp.mean(t, axis=-1, keepdims=True)
    v = jnp.mean((t - m) * (t - m), axis=-1, keepdims=True)
    return (t - m) * lax.rsqrt(v + 1e-5) * g + b


def _elu(t):
    return jnp.where(t > 0, t, jnp.exp(jnp.minimum(t, 0.0)) - 1.0)


def _pre_body(x_ref, win_ref, bin_ref, g_ref, b_ref, w0_ref, as_ref, ad_ref,
              h_ref, hw_ref, ast_ref, adt_ref):
    t = jnp.dot(x_ref[...], win_ref[...],
                preferred_element_type=jnp.float32) + bin_ref[...]
    h = _elu(_ln(t, g_ref[...], b_ref[...]))
    h_ref[...] = h
    hw = jnp.dot(h, w0_ref[...], preferred_element_type=jnp.float32)
    hw_ref[...] = hw
    ast_ref[...] = jnp.dot(hw, as_ref[...], preferred_element_type=jnp.float32)
    adt_ref[...] = jnp.dot(hw, ad_ref[...], preferred_element_type=jnp.float32)


def _mid_body(acc_ref, den_ref, bias_ref, g_ref, b_ref, r_ref, hprev_ref,
              wn_ref, as_ref, ad_ref, res_ref, hw_ref, ast_ref, adt_ref):
    a = acc_ref[0] + acc_ref[1]
    d = den_ref[0] + den_ref[1]
    db = jnp.dot(d, r_ref[...], preferred_element_type=jnp.float32) + 1e-16
    o = a / db + bias_ref[...]
    o = _elu(_ln(o, g_ref[...], b_ref[...])) + hprev_ref[...]
    res_ref[...] = o
    hw = jnp.dot(o, wn_ref[...], preferred_element_type=jnp.float32)
    hw_ref[...] = hw
    ast_ref[...] = jnp.dot(hw, as_ref[...], preferred_element_type=jnp.float32)
    adt_ref[...] = jnp.dot(hw, ad_ref[...], preferred_element_type=jnp.float32)


def _fin_body(acc_ref, den_ref, bias_ref, g_ref, b_ref, r_ref, out_ref):
    a = acc_ref[0] + acc_ref[1]
    d = den_ref[0] + den_ref[1]
    db = jnp.dot(d, r_ref[...], preferred_element_type=jnp.float32) + 1e-16
    out_ref[...] = _ln(a / db + bias_ref[...], g_ref[...], b_ref[...])


def _row_spec(width):
    return pl.BlockSpec((BL, width), lambda i: (i, 0))


def _full_spec(shape):
    return pl.BlockSpec(shape, lambda i: tuple(0 for _ in shape))


def _pre_call(x, w_in, b_in, g, b, w0, a0s, a0d):
    grid = (NPAD // BL,)
    return pl.pallas_call(
        _pre_body,
        grid=grid,
        in_specs=[
            _row_spec(IN),
            _full_spec((IN, HID)),
            _full_spec((1, HID)),
            _full_spec((1, HID)),
            _full_spec((1, HID)),
            _full_spec((HID, HID)),
            _full_spec((HID, 16)),
            _full_spec((HID, 16)),
        ],
        out_specs=[_row_spec(HID), _row_spec(HID), _row_spec(16),
                   _row_spec(16)],
        out_shape=[
            jax.ShapeDtypeStruct((NPAD, HID), jnp.float32),
            jax.ShapeDtypeStruct((NPAD, HID), jnp.float32),
            jax.ShapeDtypeStruct((NPAD, 16), jnp.float32),
            jax.ShapeDtypeStruct((NPAD, 16), jnp.float32),
        ],
    )(x, w_in, b_in, g, b, w0, a0s, a0d)


def _mid_call(acc, den, bias, g, b, rmat, hprev, wn, ans, and_, hid_next):
    grid = (NPAD // BL,)
    return pl.pallas_call(
        _mid_body,
        grid=grid,
        in_specs=[
            pl.BlockSpec((2, BL, HID), lambda i: (0, i, 0)),
            pl.BlockSpec((2, BL, 16), lambda i: (0, i, 0)),
            _full_spec((1, HID)),
            _full_spec((1, HID)),
            _full_spec((1, HID)),
            _full_spec((16, HID)),
            _row_spec(HID),
            _full_spec((HID, hid_next)),
            _full_spec((hid_next, 16)),
            _full_spec((hid_next, 16)),
        ],
        out_specs=[_row_spec(HID), _row_spec(hid_next), _row_spec(16),
                   _row_spec(16)],
        out_shape=[
            jax.ShapeDtypeStruct((NPAD, HID), jnp.float32),
            jax.ShapeDtypeStruct((NPAD, hid_next), jnp.float32),
            jax.ShapeDtypeStruct((NPAD, 16), jnp.float32),
            jax.ShapeDtypeStruct((NPAD, 16), jnp.float32),
        ],
    )(acc, den, bias, g, b, rmat, hprev, wn, ans, and_)


def _fin_call(acc, den, bias, g, b, rmat):
    grid = (NPAD // BL,)
    return pl.pallas_call(
        _fin_body,
        grid=grid,
        in_specs=[
            pl.BlockSpec((2, BL, OUT), lambda i: (0, i, 0)),
            pl.BlockSpec((2, BL, 16), lambda i: (0, i, 0)),
            _full_spec((1, OUT)),
            _full_spec((1, OUT)),
            _full_spec((1, OUT)),
            _full_spec((16, OUT)),
        ],
        out_specs=[_row_spec(OUT)],
        out_shape=[jax.ShapeDtypeStruct((NPAD, OUT), jnp.float32)],
    )(acc, den, bias, g, b, rmat)[0]


def _make_edge_pass(hid, ew):
    nreg = hid // 16
    nch = ew // CH
    assert nch % 4 == 0 and nch >= 4
    t_max = nch // 4
    rows_per = NPAD // NS
    nzc = rows_per // RC
    heads8 = hid == HID
    mesh = plsc.VectorSubcoreMesh(core_axis_name="c", subcore_axis_name="s")

    @functools.partial(
        pl.kernel,
        out_type=[
            jax.ShapeDtypeStruct((NC, NPAD, hid), jnp.float32),
            jax.ShapeDtypeStruct((NC, NPAD, 16), jnp.float32),
        ],
        mesh=mesh,
        scratch_types=(
            [pltpu.VMEM((2, CH), jnp.int32)] * 4 +
            [pltpu.VMEM((CH, hid), jnp.float32)] * 2 +
            [pltpu.VMEM((CH, 16), jnp.float32)] * 6 +
            [
                pltpu.VMEM_SHARED((NPAD, hid), jnp.float32),
                pltpu.VMEM_SHARED((NPAD, 16), jnp.float32),
            ] +
            [pltpu.SemaphoreType.DMA] * 5
        ),
        compiler_params=pltpu.CompilerParams(use_tc_tiling_on_sc=False),
    )
    def edge_pass(sd_hbm, hw_hbm, as_hbm, ad_hbm, acc_out, den_out,
                  sd0, sd1, sd2, sd3, hg_a, hg_b,
                  asg_a, asg_b, adg_a, adg_b, exu_a, exu_b,
                  acc_s, den_s, semi, semg_a, semg_b, sems_a, sems_b):
        cid = lax.axis_index("c")
        sid = lax.axis_index("s")
        wid = sid * NC + cid

        sd4 = [sd0, sd1, sd2, sd3]
        sv4 = [r.at[0] for r in sd4]
        dv4 = [r.at[1] for r in sd4]
        hg2 = [hg_a, hg_b]
        asg2 = [asg_a, asg_b]
        adg2 = [adg_a, adg_b]
        exu2 = [exu_a, exu_b]
        semg2 = [semg_a, semg_b]
        sems2 = [sems_a, sems_b]

        zero16 = jnp.zeros((16,), jnp.float32)

        def zero_row(r, _):
            for j in range(nreg):
                hg_a[r, pl.ds(j * 16, 16)] = zero16
            exu_a[r, :] = zero16
            return 0

        lax.fori_loop(0, CH, zero_row, 0)

        base_r = sid * rows_per

        def zero_spmem(k, _):
            pltpu.async_copy(hg_a.at[pl.ds(0, RC)],
                             acc_s.at[pl.ds(base_r + k * RC, RC)], semi)
            pltpu.async_copy(exu_a.at[pl.ds(0, RC)],
                             den_s.at[pl.ds(base_r + k * RC, RC)], semi)
            return 0

        def zero_wait(k, _):
            pltpu.make_async_copy(
                hg_a.at[pl.ds(0, RC)], acc_s.at[pl.ds(base_r, RC)],
                semi).wait()
            pltpu.make_async_copy(
                exu_a.at[pl.ds(0, RC)], den_s.at[pl.ds(base_r, RC)],
                semi).wait()
            return 0

        lax.fori_loop(0, nzc, zero_spmem, 0)
        lax.fori_loop(0, nzc, zero_wait, 0)
        plsc.subcore_barrier()

        bc_idx = [
            jnp.full((16,), j if heads8 else 0, jnp.int32)
            for j in range(nreg)
        ]

        base_c = wid * (ew // CH)

        def issue_idx(ci, sl):
            pltpu.async_copy(sd_hbm.at[base_c + ci], sd4[sl], semi)

        def wait_idx(sl):
            pltpu.make_async_copy(sd_hbm.at[0], sd4[sl], semi).wait()

        def issue_gath(sl, b):
            pltpu.async_copy(hw_hbm.at[sv4[sl]], hg2[b], semg2[b])
            pltpu.async_copy(as_hbm.at[sv4[sl]], asg2[b], semg2[b])
            pltpu.async_copy(ad_hbm.at[dv4[sl]], adg2[b], semg2[b])

        def wait_gath(sl, b):
            pltpu.make_async_copy(hw_hbm.at[sv4[sl]], hg2[b], semg2[b]).wait()
            pltpu.make_async_copy(
                as_hbm.at[sv4[sl]], asg2[b], semg2[b]).wait()
            pltpu.make_async_copy(
                ad_hbm.at[dv4[sl]], adg2[b], semg2[b]).wait()

        def issue_scat(sl, b):
            pltpu.async_copy(hg2[b], acc_s.at[dv4[sl]], sems2[b], add=True)
            pltpu.async_copy(exu2[b], den_s.at[dv4[sl]], sems2[b], add=True)

        def wait_scat(sl, b):
            pltpu.make_async_copy(
                hg2[b], acc_s.at[dv4[sl]], sems2[b]).wait()
            pltpu.make_async_copy(
                exu2[b], den_s.at[dv4[sl]], sems2[b]).wait()

        def compute(b):
            hg, asg, adg, exu = hg2[b], asg2[b], adg2[b], exu2[b]

            @plsc.parallel_loop(0, CH, 1, unroll=4)
            def edge_body(e):
                x = asg[e, :] + adg[e, :]
                ex = jnp.exp(jnp.maximum(x, 0.2 * x))
                exu[e, :] = ex
                for j in range(nreg):
                    bc = ex.at[bc_idx[j]].get(mode="promise_in_bounds")
                    hg[e, pl.ds(j * 16, 16)] = hg[e, pl.ds(j * 16, 16)] * bc

        issue_idx(jnp.int32(0), 0)
        issue_idx(jnp.int32(1), 1)
        wait_idx(0)
        issue_gath(0, 0)

        def body4(t, _):
            for k in range(4):
                g = 4 * t + k
                b = k % 2
                bn = (k + 1) % 2
                sl = k
                sln = (k + 1) % 4
                slp = (k - 1) % 4
                wait_gath(sl, b)
                if k < 3:
                    wait_idx(sln)
                    if k == 0:
                        @pl.when(t > 0)
                        def _():
                            wait_scat(slp, bn)
                    else:
                        wait_scat(slp, bn)
                    issue_gath(sln, bn)
                else:
                    @pl.when(t < t_max - 1)
                    def _():
                        wait_idx(sln)
                        wait_scat(slp, bn)
                        issue_gath(sln, bn)
                compute(b)
                issue_scat(sl, b)
                if k < 2:
                    issue_idx(g + 2, (k + 2) % 4)
                else:
                    @pl.when(t < t_max - 1)
                    def _():
                        issue_idx(g + 2, (k + 2) % 4)
            return 0

        lax.fori_loop(0, t_max, body4, 0)
        wait_scat(2, 0)
        wait_scat(3, 1)
        plsc.subcore_barrier()

        def copy_out(k, _):
            sl = pl.ds(base_r + k * RC, RC)
            pltpu.async_copy(acc_s.at[sl], acc_out.at[cid, sl], semi)
            pltpu.async_copy(den_s.at[sl], den_out.at[cid, sl], semi)
            return 0

        def copy_wait(k, _):
            sl = pl.ds(base_r + k * RC, RC)
            pltpu.make_async_copy(
                acc_s.at[sl], acc_out.at[cid, sl], semi).wait()
            pltpu.make_async_copy(
                den_s.at[sl], den_out.at[cid, sl], semi).wait()
            return 0

        lax.fori_loop(0, nzc, copy_out, 0)
        lax.fori_loop(0, nzc, copy_wait, 0)

    return edge_pass


def _amats(att_s, att_d, hid_in):
    z8 = jnp.zeros((HID, 8), jnp.float32)
    if hid_in == HID:
        asf = att_s.reshape(HID, 1)
        adf = att_d.reshape(HID, 1)
        amat_s = jnp.concatenate([jnp.asarray(_KRON) * asf, z8], axis=1)
        amat_d = jnp.concatenate([jnp.asarray(_KRON) * adf, z8], axis=1)
        return amat_s, amat_d
    z15 = jnp.zeros((OUT, 15), jnp.float32)
    amat_s = jnp.concatenate([att_s.reshape(OUT, 1), z15], axis=1)
    amat_d = jnp.concatenate([att_d.reshape(OUT, 1), z15], axis=1)
    return amat_s, amat_d


def kernel(x, edge_index, W_in, b_in, ln_in_g, ln_in_b, W0, as0, ad0, bc0,
           ln0_g, ln0_b, W1, as1, ad1, bc1, ln1_g, ln1_b, W2, as2, ad2, bc2,
           ln2_g, ln2_b):
    e0 = edge_index.shape[1]
    etot = e0 + N
    nch = 4 * (-(-etot // (NW * CH * 4)))
    ew = nch * CH
    ep = NW * ew
    npad_extra = NPAD - N

    loops = jnp.arange(N, dtype=jnp.int32)
    padc = ep - etot
    pad_idx = (N + (jnp.arange(padc, dtype=jnp.int32) % npad_extra))
    src_all = jnp.concatenate([edge_index[0].astype(jnp.int32), loops, pad_idx])
    dst_all = jnp.concatenate([edge_index[1].astype(jnp.int32), loops, pad_idx])
    sd_all = jnp.stack([src_all.reshape(NW * nch, CH),
                        dst_all.reshape(NW * nch, CH)], axis=1)

    xp = jnp.concatenate([x, jnp.zeros((npad_extra, IN), jnp.float32)], axis=0)

    r2 = lambda v: v.reshape(1, -1)
    r128 = jnp.asarray(_R128)
    r64 = jnp.asarray(_R64)

    edge128 = _make_edge_pass(HID, ew)
    edge64 = _make_edge_pass(OUT, ew)

    a0s, a0d = _amats(as0, ad0, HID)
    a1s, a1d = _amats(as1, ad1, HID)
    a2s, a2d = _amats(as2, ad2, OUT)

    h0, hw0, ast0, adt0 = _pre_call(xp, W_in, r2(b_in), r2(ln_in_g),
                                    r2(ln_in_b), W0, a0s, a0d)
    acc0, den0 = edge128(sd_all, hw0, ast0, adt0)
    h1, hw1, ast1, adt1 = _mid_call(acc0, den0, r2(bc0), r2(ln0_g),
                                    r2(ln0_b), r128, h0, W1, a1s, a1d, HID)
    acc1, den1 = edge128(sd_all, hw1, ast1, adt1)
    h2, hw2, ast2, adt2 = _mid_call(acc1, den1, r2(bc1), r2(ln1_g),
                                    r2(ln1_b), r128, h1, W2, a2s, a2d, OUT)
    acc2, den2 = edge64(sd_all, hw2, ast2, adt2)
    out = _fin_call(acc2, den2, r2(bc2), r2(ln2_g), r2(ln2_b), r64)
    return out[:N]


# speedup vs baseline: 1.1436x; 1.0370x over previous
import functools

import jax
import jax.numpy as jnp
import numpy as np
from jax import lax
from jax.experimental import pallas as pl
from jax.experimental.pallas import tpu as pltpu
from jax.experimental.pallas import tpu_sc as plsc

N = 10000
NPAD = 10240
IN = 128
H = 8
C = 16
HID = H * C
OUT = 64

NC = 2
NS = 16
NW = NC * NS
CH = 104
RC = 64
BL = 1024

_R128 = np.zeros((16, HID), np.float32)
for _h in range(H):
    _R128[_h, _h * C:(_h + 1) * C] = 1.0
_R64 = np.zeros((16, OUT), np.float32)
_R64[0, :] = 1.0
_KRON = np.kron(np.eye(H, dtype=np.float32), np.ones((C, 1), np.float32))


def _ln(t, g, b):
    m = jnp.mean(t, axis=-1, keepdims=True)
    v = jnp.mean((t - m) * (t - m), axis=-1, keepdims=True)
    return (t - m) * lax.rsqrt(v + 1e-5) * g + b


def _elu(t):
    return jnp.where(t > 0, t, jnp.exp(jnp.minimum(t, 0.0)) - 1.0)


def _pre_body(x_ref, win_ref, bin_ref, g_ref, b_ref, w0_ref, as_ref, ad_ref,
              h_ref, hw_ref, ast_ref, adt_ref):
    t = jnp.dot(x_ref[...], win_ref[...],
                preferred_element_type=jnp.float32) + bin_ref[...]
    h = _elu(_ln(t, g_ref[...], b_ref[...]))
    h_ref[...] = h
    hw = jnp.dot(h, w0_ref[...], preferred_element_type=jnp.float32)
    hw_ref[...] = hw
    ast_ref[...] = jnp.dot(hw, as_ref[...], preferred_element_type=jnp.float32)
    adt_ref[...] = jnp.dot(hw, ad_ref[...], preferred_element_type=jnp.float32)


def _mid_body(acc_ref, den_ref, bias_ref, g_ref, b_ref, r_ref, hprev_ref,
              wn_ref, as_ref, ad_ref, res_ref, hw_ref, ast_ref, adt_ref):
    a = acc_ref[0] + acc_ref[1]
    d = den_ref[0] + den_ref[1]
    db = jnp.dot(d, r_ref[...], preferred_element_type=jnp.float32) + 1e-16
    o = a / db + bias_ref[...]
    o = _elu(_ln(o, g_ref[...], b_ref[...])) + hprev_ref[...]
    res_ref[...] = o
    hw = jnp.dot(o, wn_ref[...], preferred_element_type=jnp.float32)
    hw_ref[...] = hw
    ast_ref[...] = jnp.dot(hw, as_ref[...], preferred_element_type=jnp.float32)
    adt_ref[...] = jnp.dot(hw, ad_ref[...], preferred_element_type=jnp.float32)


def _fin_body(acc_ref, den_ref, bias_ref, g_ref, b_ref, r_ref, out_ref):
    a = acc_ref[0] + acc_ref[1]
    d = den_ref[0] + den_ref[1]
    db = jnp.dot(d, r_ref[...], preferred_element_type=jnp.float32) + 1e-16
    out_ref[...] = _ln(a / db + bias_ref[...], g_ref[...], b_ref[...])


def _row_spec(width):
    return pl.BlockSpec((BL, width), lambda i: (i, 0))


def _full_spec(shape):
    return pl.BlockSpec(shape, lambda i: tuple(0 for _ in shape))


def _pre_call(x, w_in, b_in, g, b, w0, a0s, a0d):
    grid = (NPAD // BL,)
    return pl.pallas_call(
        _pre_body,
        grid=grid,
        in_specs=[
            _row_spec(IN),
            _full_spec((IN, HID)),
            _full_spec((1, HID)),
            _full_spec((1, HID)),
            _full_spec((1, HID)),
            _full_spec((HID, HID)),
            _full_spec((HID, 16)),
            _full_spec((HID, 16)),
        ],
        out_specs=[_row_spec(HID), _row_spec(HID), _row_spec(16),
                   _row_spec(16)],
        out_shape=[
            jax.ShapeDtypeStruct((NPAD, HID), jnp.float32),
            jax.ShapeDtypeStruct((NPAD, HID), jnp.float32),
            jax.ShapeDtypeStruct((NPAD, 16), jnp.float32),
            jax.ShapeDtypeStruct((NPAD, 16), jnp.float32),
        ],
    )(x, w_in, b_in, g, b, w0, a0s, a0d)


def _mid_call(acc, den, bias, g, b, rmat, hprev, wn, ans, and_, hid_next):
    grid = (NPAD // BL,)
    return pl.pallas_call(
        _mid_body,
        grid=grid,
        in_specs=[
            pl.BlockSpec((2, BL, HID), lambda i: (0, i, 0)),
            pl.BlockSpec((2, BL, 16), lambda i: (0, i, 0)),
            _full_spec((1, HID)),
            _full_spec((1, HID)),
            _full_spec((1, HID)),
            _full_spec((16, HID)),
            _row_spec(HID),
            _full_spec((HID, hid_next)),
            _full_spec((hid_next, 16)),
            _full_spec((hid_next, 16)),
        ],
        out_specs=[_row_spec(HID), _row_spec(hid_next), _row_spec(16),
                   _row_spec(16)],
        out_shape=[
            jax.ShapeDtypeStruct((NPAD, HID), jnp.float32),
            jax.ShapeDtypeStruct((NPAD, hid_next), jnp.float32),
            jax.ShapeDtypeStruct((NPAD, 16), jnp.float32),
            jax.ShapeDtypeStruct((NPAD, 16), jnp.float32),
        ],
    )(acc, den, bias, g, b, rmat, hprev, wn, ans, and_)


def _fin_call(acc, den, bias, g, b, rmat):
    grid = (NPAD // BL,)
    return pl.pallas_call(
        _fin_body,
        grid=grid,
        in_specs=[
            pl.BlockSpec((2, BL, OUT), lambda i: (0, i, 0)),
            pl.BlockSpec((2, BL, 16), lambda i: (0, i, 0)),
            _full_spec((1, OUT)),
            _full_spec((1, OUT)),
            _full_spec((1, OUT)),
            _full_spec((16, OUT)),
        ],
        out_specs=[_row_spec(OUT)],
        out_shape=[jax.ShapeDtypeStruct((NPAD, OUT), jnp.float32)],
    )(acc, den, bias, g, b, rmat)[0]


def _make_edge_pass(hid, ew):
    nreg = hid // 16
    nch = ew // CH
    assert nch % 4 == 0 and nch >= 4
    t_max = nch // 4
    rows_per = NPAD // NS
    nzc = rows_per // RC
    heads8 = hid == HID
    mesh = plsc.VectorSubcoreMesh(core_axis_name="c", subcore_axis_name="s")

    @functools.partial(
        pl.kernel,
        out_type=[
            jax.ShapeDtypeStruct((NC, NPAD, hid), jnp.float32),
            jax.ShapeDtypeStruct((NC, NPAD, 16), jnp.float32),
        ],
        mesh=mesh,
        scratch_types=(
            [pltpu.VMEM((2, CH), jnp.int32)] * 4 +
            [pltpu.VMEM((CH, hid), jnp.float32)] * 2 +
            [pltpu.VMEM((CH, 16), jnp.float32)] * 6 +
            [
                pltpu.VMEM_SHARED((NPAD, hid), jnp.float32),
                pltpu.VMEM_SHARED((NPAD, 16), jnp.float32),
            ] +
            [pltpu.SemaphoreType.DMA] * 5
        ),
        compiler_params=pltpu.CompilerParams(use_tc_tiling_on_sc=False),
    )
    def edge_pass(sd_hbm, hw_hbm, as_hbm, ad_hbm, acc_out, den_out,
                  sd0, sd1, sd2, sd3, hg_a, hg_b,
                  asg_a, asg_b, adg_a, adg_b, exu_a, exu_b,
                  acc_s, den_s, semi, semg_a, semg_b, sems_a, sems_b):
        cid = lax.axis_index("c")
        sid = lax.axis_index("s")
        wid = sid * NC + cid

        sd4 = [sd0, sd1, sd2, sd3]
        sv4 = [r.at[0] for r in sd4]
        dv4 = [r.at[1] for r in sd4]
        hg2 = [hg_a, hg_b]
        asg2 = [asg_a, asg_b]
        adg2 = [adg_a, adg_b]
        exu2 = [exu_a, exu_b]
        semg2 = [semg_a, semg_b]
        sems2 = [sems_a, sems_b]

        zero16 = jnp.zeros((16,), jnp.float32)

        def zero_row(r, _):
            for j in range(nreg):
                hg_a[r, pl.ds(j * 16, 16)] = zero16
            exu_a[r, :] = zero16
            return 0

        lax.fori_loop(0, CH, zero_row, 0)

        base_r = sid * rows_per

        def zero_spmem(k, _):
            pltpu.async_copy(hg_a.at[pl.ds(0, RC)],
                             acc_s.at[pl.ds(base_r + k * RC, RC)], semi)
            pltpu.async_copy(exu_a.at[pl.ds(0, RC)],
                             den_s.at[pl.ds(base_r + k * RC, RC)], semi)
            return 0

        def zero_wait(k, _):
            pltpu.make_async_copy(
                hg_a.at[pl.ds(0, RC)], acc_s.at[pl.ds(base_r, RC)],
                semi).wait()
            pltpu.make_async_copy(
                exu_a.at[pl.ds(0, RC)], den_s.at[pl.ds(base_r, RC)],
                semi).wait()
            return 0

        lax.fori_loop(0, nzc, zero_spmem, 0)
        lax.fori_loop(0, nzc, zero_wait, 0)
        plsc.subcore_barrier()

        bc_idx = [
            jnp.full((16,), j if heads8 else 0, jnp.int32)
            for j in range(nreg)
        ]

        base_c = wid * (ew // CH)

        def issue_idx(ci, sl):
            pltpu.async_copy(sd_hbm.at[base_c + ci], sd4[sl], semi)

        def wait_idx(sl):
            pltpu.make_async_copy(sd_hbm.at[0], sd4[sl], semi).wait()

        def issue_gath(sl, b):
            pltpu.async_copy(hw_hbm.at[sv4[sl]], hg2[b], semg2[b])
            pltpu.async_copy(as_hbm.at[sv4[sl]], asg2[b], semg2[b])
            pltpu.async_copy(ad_hbm.at[dv4[sl]], adg2[b], semg2[b])

        def wait_gath(sl, b):
            pltpu.make_async_copy(hw_hbm.at[sv4[sl]], hg2[b], semg2[b]).wait()
            pltpu.make_async_copy(
                as_hbm.at[sv4[sl]], asg2[b], semg2[b]).wait()
            pltpu.make_async_copy(
                ad_hbm.at[dv4[sl]], adg2[b], semg2[b]).wait()

        def issue_scat(sl, b):
            pltpu.async_copy(hg2[b], acc_s.at[dv4[sl]], sems2[b], add=True)
            pltpu.async_copy(exu2[b], den_s.at[dv4[sl]], sems2[b], add=True)

        def wait_scat(sl, b):
            pltpu.make_async_copy(
                hg2[b], acc_s.at[dv4[sl]], sems2[b]).wait()
            pltpu.make_async_copy(
                exu2[b], den_s.at[dv4[sl]], sems2[b]).wait()

        def compute(b):
            hg, asg, adg, exu = hg2[b], asg2[b], adg2[b], exu2[b]

            @plsc.parallel_loop(0, CH, 1, unroll=4)
            def edge_body(e):
                x = asg[e, :] + adg[e, :]
                ex = jnp.exp(jnp.maximum(x, 0.2 * x))
                exu[e, :] = ex
                for j in range(nreg):
                    bc = ex.at[bc_idx[j]].get(mode="promise_in_bounds")
                    hg[e, pl.ds(j * 16, 16)] = hg[e, pl.ds(j * 16, 16)] * bc

        issue_idx(jnp.int32(0), 0)
        issue_idx(jnp.int32(1), 1)
        wait_idx(0)
        issue_gath(0, 0)

        def body4(t, _):
            for k in range(4):
                g = 4 * t + k
                b = k % 2
                bn = (k + 1) % 2
                sl = k
                sln = (k + 1) % 4
                slp = (k - 1) % 4
                wait_gath(sl, b)
                if k < 3:
                    wait_idx(sln)
                    if k == 0:
                        @pl.when(t > 0)
                        def _():
                            wait_scat(slp, bn)
                    else:
                        wait_scat(slp, bn)
                    issue_gath(sln, bn)
                else:
                    @pl.when(t < t_max - 1)
                    def _():
                        wait_idx(sln)
                        wait_scat(slp, bn)
                        issue_gath(sln, bn)
                compute(b)
                issue_scat(sl, b)
                if k < 2:
                    issue_idx(g + 2, (k + 2) % 4)
                else:
                    @pl.when(t < t_max - 1)
                    def _():
                        issue_idx(g + 2, (k + 2) % 4)
            return 0

        lax.fori_loop(0, t_max, body4, 0)
        wait_scat(2, 0)
        wait_scat(3, 1)
        plsc.subcore_barrier()

        def copy_out(k, _):
            sl = pl.ds(base_r + k * RC, RC)
            pltpu.async_copy(acc_s.at[sl], acc_out.at[cid, sl], semi)
            pltpu.async_copy(den_s.at[sl], den_out.at[cid, sl], semi)
            return 0

        def copy_wait(k, _):
            sl = pl.ds(base_r + k * RC, RC)
            pltpu.make_async_copy(
                acc_s.at[sl], acc_out.at[cid, sl], semi).wait()
            pltpu.make_async_copy(
                den_s.at[sl], den_out.at[cid, sl], semi).wait()
            return 0

        lax.fori_loop(0, nzc, copy_out, 0)
        lax.fori_loop(0, nzc, copy_wait, 0)

    return edge_pass


def _amats(att_s, att_d, hid_in):
    z8 = jnp.zeros((HID, 8), jnp.float32)
    if hid_in == HID:
        asf = att_s.reshape(HID, 1)
        adf = att_d.reshape(HID, 1)
        amat_s = jnp.concatenate([jnp.asarray(_KRON) * asf, z8], axis=1)
        amat_d = jnp.concatenate([jnp.asarray(_KRON) * adf, z8], axis=1)
        return amat_s, amat_d
    z15 = jnp.zeros((OUT, 15), jnp.float32)
    amat_s = jnp.concatenate([att_s.reshape(OUT, 1), z15], axis=1)
    amat_d = jnp.concatenate([att_d.reshape(OUT, 1), z15], axis=1)
    return amat_s, amat_d


def kernel(x, edge_index, W_in, b_in, ln_in_g, ln_in_b, W0, as0, ad0, bc0,
           ln0_g, ln0_b, W1, as1, ad1, bc1, ln1_g, ln1_b, W2, as2, ad2, bc2,
           ln2_g, ln2_b):
    e0 = edge_index.shape[1]
    etot = e0 + N
    nch = 4 * (-(-etot // (NW * CH * 4)))
    ew = nch * CH
    ep = NW * ew
    npad_extra = NPAD - N

    loops = jnp.arange(N, dtype=jnp.int32)
    padc = ep - etot
    pad_idx = (N + (jnp.arange(padc, dtype=jnp.int32) % npad_extra))
    src_all = jnp.concatenate([edge_index[0].astype(jnp.int32), loops, pad_idx])
    dst_all = jnp.concatenate([edge_index[1].astype(jnp.int32), loops, pad_idx])
    sd_all = jnp.stack([src_all.reshape(NW * nch, CH),
                        dst_all.reshape(NW * nch, CH)], axis=1)

    xp = jnp.concatenate([x, jnp.zeros((npad_extra, IN), jnp.float32)], axis=0)

    r2 = lambda v: v.reshape(1, -1)
    r128 = jnp.asarray(_R128)
    r64 = jnp.asarray(_R64)

    edge128 = _make_edge_pass(HID, ew)
    edge64 = _make_edge_pass(OUT, ew)

    a0s, a0d = _amats(as0, ad0, HID)
    a1s, a1d = _amats(as1, ad1, HID)
    a2s, a2d = _amats(as2, ad2, OUT)

    h0, hw0, ast0, adt0 = _pre_call(xp, W_in, r2(b_in), r2(ln_in_g),
                                    r2(ln_in_b), W0, a0s, a0d)
    acc0, den0 = edge128(sd_all, hw0, ast0, adt0)
    h1, hw1, ast1, adt1 = _mid_call(acc0, den0, r2(bc0), r2(ln0_g),
                                    r2(ln0_b), r128, h0, W1, a1s, a1d, HID)
    acc1, den1 = edge128(sd_all, hw1, ast1, adt1)
    h2, hw2, ast2, adt2 = _mid_call(acc1, den1, r2(bc1), r2(ln1_g),
                                    r2(ln1_b), r128, h1, W2, a2s, a2d, OUT)
    acc2, den2 = edge64(sd_all, hw2, ast2, adt2)
    out = _fin_call(acc2, den2, r2(bc2), r2(ln2_g), r2(ln2_b), r64)
    return out[:N]


# speedup vs baseline: 1.1643x; 1.0181x over previous
import functools

import jax
import jax.numpy as jnp
import numpy as np
from jax import lax
from jax.experimental import pallas as pl
from jax.experimental.pallas import tpu as pltpu
from jax.experimental.pallas import tpu_sc as plsc

N = 10000
NPAD = 10240
IN = 128
H = 8
C = 16
HID = H * C
OUT = 64

NC = 2
NS = 16
NW = NC * NS
CH = 104
RC = 64
BL = 2560

_R128 = np.zeros((16, HID), np.float32)
for _h in range(H):
    _R128[_h, _h * C:(_h + 1) * C] = 1.0
_R64 = np.zeros((16, OUT), np.float32)
_R64[0, :] = 1.0
_KRON = np.kron(np.eye(H, dtype=np.float32), np.ones((C, 1), np.float32))


def _ln(t, g, b):
    m = jnp.mean(t, axis=-1, keepdims=True)
    v = jnp.mean((t - m) * (t - m), axis=-1, keepdims=True)
    return (t - m) * lax.rsqrt(v + 1e-5) * g + b


def _elu(t):
    return jnp.where(t > 0, t, jnp.exp(jnp.minimum(t, 0.0)) - 1.0)


def _pre_body(x_ref, win_ref, bin_ref, g_ref, b_ref, w0_ref, as_ref, ad_ref,
              h_ref, hw_ref, ast_ref, adt_ref):
    t = jnp.dot(x_ref[...], win_ref[...],
                preferred_element_type=jnp.float32) + bin_ref[...]
    h = _elu(_ln(t, g_ref[...], b_ref[...]))
    h_ref[...] = h
    hw = jnp.dot(h, w0_ref[...], preferred_element_type=jnp.float32)
    hw_ref[...] = hw
    ast_ref[...] = jnp.dot(hw, as_ref[...], preferred_element_type=jnp.float32)
    adt_ref[...] = jnp.dot(hw, ad_ref[...], preferred_element_type=jnp.float32)


def _mid_body(acc_ref, den_ref, bias_ref, g_ref, b_ref, r_ref, hprev_ref,
              wn_ref, as_ref, ad_ref, res_ref, hw_ref, ast_ref, adt_ref):
    a = acc_ref[0] + acc_ref[1]
    d = den_ref[0] + den_ref[1]
    db = jnp.dot(d, r_ref[...], preferred_element_type=jnp.float32) + 1e-16
    o = a / db + bias_ref[...]
    o = _elu(_ln(o, g_ref[...], b_ref[...])) + hprev_ref[...]
    res_ref[...] = o
    hw = jnp.dot(o, wn_ref[...], preferred_element_type=jnp.float32)
    hw_ref[...] = hw
    ast_ref[...] = jnp.dot(hw, as_ref[...], preferred_element_type=jnp.float32)
    adt_ref[...] = jnp.dot(hw, ad_ref[...], preferred_element_type=jnp.float32)


def _fin_body(acc_ref, den_ref, bias_ref, g_ref, b_ref, r_ref, out_ref):
    a = acc_ref[0] + acc_ref[1]
    d = den_ref[0] + den_ref[1]
    db = jnp.dot(d, r_ref[...], preferred_element_type=jnp.float32) + 1e-16
    out_ref[...] = _ln(a / db + bias_ref[...], g_ref[...], b_ref[...])


def _row_spec(width):
    return pl.BlockSpec((BL, width), lambda i: (i, 0))


def _full_spec(shape):
    return pl.BlockSpec(shape, lambda i: tuple(0 for _ in shape))


def _pre_call(x, w_in, b_in, g, b, w0, a0s, a0d):
    grid = (NPAD // BL,)
    return pl.pallas_call(
        _pre_body,
        grid=grid,
        in_specs=[
            _row_spec(IN),
            _full_spec((IN, HID)),
            _full_spec((1, HID)),
            _full_spec((1, HID)),
            _full_spec((1, HID)),
            _full_spec((HID, HID)),
            _full_spec((HID, 16)),
            _full_spec((HID, 16)),
        ],
        out_specs=[_row_spec(HID), _row_spec(HID), _row_spec(16),
                   _row_spec(16)],
        out_shape=[
            jax.ShapeDtypeStruct((NPAD, HID), jnp.float32),
            jax.ShapeDtypeStruct((NPAD, HID), jnp.float32),
            jax.ShapeDtypeStruct((NPAD, 16), jnp.float32),
            jax.ShapeDtypeStruct((NPAD, 16), jnp.float32),
        ],
    )(x, w_in, b_in, g, b, w0, a0s, a0d)


def _mid_call(acc, den, bias, g, b, rmat, hprev, wn, ans, and_, hid_next):
    grid = (NPAD // BL,)
    return pl.pallas_call(
        _mid_body,
        grid=grid,
        in_specs=[
            pl.BlockSpec((2, BL, HID), lambda i: (0, i, 0)),
            pl.BlockSpec((2, BL, 16), lambda i: (0, i, 0)),
            _full_spec((1, HID)),
            _full_spec((1, HID)),
            _full_spec((1, HID)),
            _full_spec((16, HID)),
            _row_spec(HID),
            _full_spec((HID, hid_next)),
            _full_spec((hid_next, 16)),
            _full_spec((hid_next, 16)),
        ],
        out_specs=[_row_spec(HID), _row_spec(hid_next), _row_spec(16),
                   _row_spec(16)],
        out_shape=[
            jax.ShapeDtypeStruct((NPAD, HID), jnp.float32),
            jax.ShapeDtypeStruct((NPAD, hid_next), jnp.float32),
            jax.ShapeDtypeStruct((NPAD, 16), jnp.float32),
            jax.ShapeDtypeStruct((NPAD, 16), jnp.float32),
        ],
    )(acc, den, bias, g, b, rmat, hprev, wn, ans, and_)


def _fin_call(acc, den, bias, g, b, rmat):
    grid = (NPAD // BL,)
    return pl.pallas_call(
        _fin_body,
        grid=grid,
        in_specs=[
            pl.BlockSpec((2, BL, OUT), lambda i: (0, i, 0)),
            pl.BlockSpec((2, BL, 16), lambda i: (0, i, 0)),
            _full_spec((1, OUT)),
            _full_spec((1, OUT)),
            _full_spec((1, OUT)),
            _full_spec((16, OUT)),
        ],
        out_specs=[_row_spec(OUT)],
        out_shape=[jax.ShapeDtypeStruct((NPAD, OUT), jnp.float32)],
    )(acc, den, bias, g, b, rmat)[0]


def _make_edge_pass(hid, ew):
    nreg = hid // 16
    nch = ew // CH
    assert nch % 4 == 0 and nch >= 4
    t_max = nch // 4
    rows_per = NPAD // NS
    nzc = rows_per // RC
    heads8 = hid == HID
    mesh = plsc.VectorSubcoreMesh(core_axis_name="c", subcore_axis_name="s")

    @functools.partial(
        pl.kernel,
        out_type=[
            jax.ShapeDtypeStruct((NC, NPAD, hid), jnp.float32),
            jax.ShapeDtypeStruct((NC, NPAD, 16), jnp.float32),
        ],
        mesh=mesh,
        scratch_types=(
            [pltpu.VMEM((2, CH), jnp.int32)] * 4 +
            [pltpu.VMEM((CH, hid), jnp.float32)] * 2 +
            [pltpu.VMEM((CH, 16), jnp.float32)] * 6 +
            [
                pltpu.VMEM_SHARED((NPAD, hid), jnp.float32),
                pltpu.VMEM_SHARED((NPAD, 16), jnp.float32),
            ] +
            [pltpu.SemaphoreType.DMA] * 5
        ),
        compiler_params=pltpu.CompilerParams(use_tc_tiling_on_sc=False),
    )
    def edge_pass(sd_hbm, hw_hbm, as_hbm, ad_hbm, acc_out, den_out,
                  sd0, sd1, sd2, sd3, hg_a, hg_b,
                  asg_a, asg_b, adg_a, adg_b, exu_a, exu_b,
                  acc_s, den_s, semi, semg_a, semg_b, sems_a, sems_b):
        cid = lax.axis_index("c")
        sid = lax.axis_index("s")
        wid = sid * NC + cid

        sd4 = [sd0, sd1, sd2, sd3]
        sv4 = [r.at[0] for r in sd4]
        dv4 = [r.at[1] for r in sd4]
        hg2 = [hg_a, hg_b]
        asg2 = [asg_a, asg_b]
        adg2 = [adg_a, adg_b]
        exu2 = [exu_a, exu_b]
        semg2 = [semg_a, semg_b]
        sems2 = [sems_a, sems_b]

        zero16 = jnp.zeros((16,), jnp.float32)

        def zero_row(r, _):
            for j in range(nreg):
                hg_a[r, pl.ds(j * 16, 16)] = zero16
            exu_a[r, :] = zero16
            return 0

        lax.fori_loop(0, CH, zero_row, 0)

        base_r = sid * rows_per

        def zero_spmem(k, _):
            pltpu.async_copy(hg_a.at[pl.ds(0, RC)],
                             acc_s.at[pl.ds(base_r + k * RC, RC)], semi)
            pltpu.async_copy(exu_a.at[pl.ds(0, RC)],
                             den_s.at[pl.ds(base_r + k * RC, RC)], semi)
            return 0

        def zero_wait(k, _):
            pltpu.make_async_copy(
                hg_a.at[pl.ds(0, RC)], acc_s.at[pl.ds(base_r, RC)],
                semi).wait()
            pltpu.make_async_copy(
                exu_a.at[pl.ds(0, RC)], den_s.at[pl.ds(base_r, RC)],
                semi).wait()
            return 0

        lax.fori_loop(0, nzc, zero_spmem, 0)
        lax.fori_loop(0, nzc, zero_wait, 0)
        plsc.subcore_barrier()

        bc_idx = [
            jnp.full((16,), j if heads8 else 0, jnp.int32)
            for j in range(nreg)
        ]

        base_c = wid * (ew // CH)

        def issue_idx(ci, sl):
            pltpu.async_copy(sd_hbm.at[base_c + ci], sd4[sl], semi)

        def wait_idx(sl):
            pltpu.make_async_copy(sd_hbm.at[0], sd4[sl], semi).wait()

        def issue_gath(sl, b):
            pltpu.async_copy(hw_hbm.at[sv4[sl]], hg2[b], semg2[b])
            pltpu.async_copy(as_hbm.at[sv4[sl]], asg2[b], semg2[b])
            pltpu.async_copy(ad_hbm.at[dv4[sl]], adg2[b], semg2[b])

        def wait_gath(sl, b):
            pltpu.make_async_copy(hw_hbm.at[sv4[sl]], hg2[b], semg2[b]).wait()
            pltpu.make_async_copy(
                as_hbm.at[sv4[sl]], asg2[b], semg2[b]).wait()
            pltpu.make_async_copy(
                ad_hbm.at[dv4[sl]], adg2[b], semg2[b]).wait()

        def issue_scat(sl, b):
            pltpu.async_copy(hg2[b], acc_s.at[dv4[sl]], sems2[b], add=True)
            pltpu.async_copy(exu2[b], den_s.at[dv4[sl]], sems2[b], add=True)

        def wait_scat(sl, b):
            pltpu.make_async_copy(
                hg2[b], acc_s.at[dv4[sl]], sems2[b]).wait()
            pltpu.make_async_copy(
                exu2[b], den_s.at[dv4[sl]], sems2[b]).wait()

        def compute(b):
            hg, asg, adg, exu = hg2[b], asg2[b], adg2[b], exu2[b]

            @plsc.parallel_loop(0, CH, 1, unroll=4)
            def edge_body(e):
                x = asg[e, :] + adg[e, :]
                ex = jnp.exp(jnp.maximum(x, 0.2 * x))
                exu[e, :] = ex
                for j in range(nreg):
                    bc = ex.at[bc_idx[j]].get(mode="promise_in_bounds")
                    hg[e, pl.ds(j * 16, 16)] = hg[e, pl.ds(j * 16, 16)] * bc

        issue_idx(jnp.int32(0), 0)
        issue_idx(jnp.int32(1), 1)
        wait_idx(0)
        issue_gath(0, 0)

        def body4(t, _):
            for k in range(4):
                g = 4 * t + k
                b = k % 2
                bn = (k + 1) % 2
                sl = k
                sln = (k + 1) % 4
                slp = (k - 1) % 4
                wait_gath(sl, b)
                if k < 3:
                    wait_idx(sln)
                    if k == 0:
                        @pl.when(t > 0)
                        def _():
                            wait_scat(slp, bn)
                    else:
                        wait_scat(slp, bn)
                    issue_gath(sln, bn)
                else:
                    @pl.when(t < t_max - 1)
                    def _():
                        wait_idx(sln)
                        wait_scat(slp, bn)
                        issue_gath(sln, bn)
                compute(b)
                issue_scat(sl, b)
                if k < 2:
                    issue_idx(g + 2, (k + 2) % 4)
                else:
                    @pl.when(t < t_max - 1)
                    def _():
                        issue_idx(g + 2, (k + 2) % 4)
            return 0

        lax.fori_loop(0, t_max, body4, 0)
        wait_scat(2, 0)
        wait_scat(3, 1)
        plsc.subcore_barrier()

        def copy_out(k, _):
            sl = pl.ds(base_r + k * RC, RC)
            pltpu.async_copy(acc_s.at[sl], acc_out.at[cid, sl], semi)
            pltpu.async_copy(den_s.at[sl], den_out.at[cid, sl], semi)
            return 0

        def copy_wait(k, _):
            sl = pl.ds(base_r + k * RC, RC)
            pltpu.make_async_copy(
                acc_s.at[sl], acc_out.at[cid, sl], semi).wait()
            pltpu.make_async_copy(
                den_s.at[sl], den_out.at[cid, sl], semi).wait()
            return 0

        lax.fori_loop(0, nzc, copy_out, 0)
        lax.fori_loop(0, nzc, copy_wait, 0)

    return edge_pass


def _amats(att_s, att_d, hid_in):
    z8 = jnp.zeros((HID, 8), jnp.float32)
    if hid_in == HID:
        asf = att_s.reshape(HID, 1)
        adf = att_d.reshape(HID, 1)
        amat_s = jnp.concatenate([jnp.asarray(_KRON) * asf, z8], axis=1)
        amat_d = jnp.concatenate([jnp.asarray(_KRON) * adf, z8], axis=1)
        return amat_s, amat_d
    z15 = jnp.zeros((OUT, 15), jnp.float32)
    amat_s = jnp.concatenate([att_s.reshape(OUT, 1), z15], axis=1)
    amat_d = jnp.concatenate([att_d.reshape(OUT, 1), z15], axis=1)
    return amat_s, amat_d


def kernel(x, edge_index, W_in, b_in, ln_in_g, ln_in_b, W0, as0, ad0, bc0,
           ln0_g, ln0_b, W1, as1, ad1, bc1, ln1_g, ln1_b, W2, as2, ad2, bc2,
           ln2_g, ln2_b):
    e0 = edge_index.shape[1]
    etot = e0 + N
    nch = 4 * (-(-etot // (NW * CH * 4)))
    ew = nch * CH
    ep = NW * ew
    npad_extra = NPAD - N

    loops = jnp.arange(N, dtype=jnp.int32)
    padc = ep - etot
    pad_idx = (N + (jnp.arange(padc, dtype=jnp.int32) % npad_extra))
    src_all = jnp.concatenate([edge_index[0].astype(jnp.int32), loops, pad_idx])
    dst_all = jnp.concatenate([edge_index[1].astype(jnp.int32), loops, pad_idx])
    sd_all = jnp.stack([src_all.reshape(NW * nch, CH),
                        dst_all.reshape(NW * nch, CH)], axis=1)

    xp = jnp.concatenate([x, jnp.zeros((npad_extra, IN), jnp.float32)], axis=0)

    r2 = lambda v: v.reshape(1, -1)
    r128 = jnp.asarray(_R128)
    r64 = jnp.asarray(_R64)

    edge128 = _make_edge_pass(HID, ew)
    edge64 = _make_edge_pass(OUT, ew)

    a0s, a0d = _amats(as0, ad0, HID)
    a1s, a1d = _amats(as1, ad1, HID)
    a2s, a2d = _amats(as2, ad2, OUT)

    h0, hw0, ast0, adt0 = _pre_call(xp, W_in, r2(b_in), r2(ln_in_g),
                                    r2(ln_in_b), W0, a0s, a0d)
    acc0, den0 = edge128(sd_all, hw0, ast0, adt0)
    h1, hw1, ast1, adt1 = _mid_call(acc0, den0, r2(bc0), r2(ln0_g),
                                    r2(ln0_b), r128, h0, W1, a1s, a1d, HID)
    acc1, den1 = edge128(sd_all, hw1, ast1, adt1)
    h2, hw2, ast2, adt2 = _mid_call(acc1, den1, r2(bc1), r2(ln1_g),
                                    r2(ln1_b), r128, h1, W2, a2s, a2d, OUT)
    acc2, den2 = edge64(sd_all, hw2, ast2, adt2)
    out = _fin_call(acc2, den2, r2(bc2), r2(ln2_g), r2(ln2_b), r64)
    return out[:N]


# speedup vs baseline: 1.1646x; 1.0003x over previous
import functools

import jax
import jax.numpy as jnp
import numpy as np
from jax import lax
from jax.experimental import pallas as pl
from jax.experimental.pallas import tpu as pltpu
from jax.experimental.pallas import tpu_sc as plsc

N = 10000
NPAD = 10240
IN = 128
H = 8
C = 16
HID = H * C
OUT = 64

NC = 2
NS = 16
NW = NC * NS
CH = 104
RC = 64
BL = 2560

_R128 = np.zeros((16, HID), np.float32)
for _h in range(H):
    _R128[_h, _h * C:(_h + 1) * C] = 1.0
_R64 = np.zeros((16, OUT), np.float32)
_R64[0, :] = 1.0
_KRON = np.kron(np.eye(H, dtype=np.float32), np.ones((C, 1), np.float32))


def _ln(t, g, b):
    m = jnp.mean(t, axis=-1, keepdims=True)
    v = jnp.mean((t - m) * (t - m), axis=-1, keepdims=True)
    return (t - m) * lax.rsqrt(v + 1e-5) * g + b


def _elu(t):
    return jnp.where(t > 0, t, jnp.exp(jnp.minimum(t, 0.0)) - 1.0)


def _pre_body(x_ref, win_ref, bin_ref, g_ref, b_ref, w0_ref, as_ref, ad_ref,
              h_ref, hw_ref, ast_ref, adt_ref):
    t = jnp.dot(x_ref[...], win_ref[...],
                preferred_element_type=jnp.float32) + bin_ref[...]
    h = _elu(_ln(t, g_ref[...], b_ref[...]))
    h_ref[...] = h
    hw = jnp.dot(h, w0_ref[...], preferred_element_type=jnp.float32)
    hw_ref[...] = hw
    ast_ref[...] = jnp.dot(hw, as_ref[...], preferred_element_type=jnp.float32)
    adt_ref[...] = jnp.dot(hw, ad_ref[...], preferred_element_type=jnp.float32)


def _mid_body(acc_ref, den_ref, bias_ref, g_ref, b_ref, r_ref, hprev_ref,
              wn_ref, as_ref, ad_ref, res_ref, hw_ref, ast_ref, adt_ref):
    a = acc_ref[0] + acc_ref[1]
    d = den_ref[0] + den_ref[1]
    db = jnp.dot(d, r_ref[...], preferred_element_type=jnp.float32) + 1e-16
    o = a / db + bias_ref[...]
    o = _elu(_ln(o, g_ref[...], b_ref[...])) + hprev_ref[...]
    res_ref[...] = o
    hw = jnp.dot(o, wn_ref[...], preferred_element_type=jnp.float32)
    hw_ref[...] = hw
    ast_ref[...] = jnp.dot(hw, as_ref[...], preferred_element_type=jnp.float32)
    adt_ref[...] = jnp.dot(hw, ad_ref[...], preferred_element_type=jnp.float32)


def _fin_body(acc_ref, den_ref, bias_ref, g_ref, b_ref, r_ref, out_ref):
    a = acc_ref[0] + acc_ref[1]
    d = den_ref[0] + den_ref[1]
    db = jnp.dot(d, r_ref[...], preferred_element_type=jnp.float32) + 1e-16
    out_ref[...] = _ln(a / db + bias_ref[...], g_ref[...], b_ref[...])


def _row_spec(width):
    return pl.BlockSpec((BL, width), lambda i: (i, 0))


def _full_spec(shape):
    return pl.BlockSpec(shape, lambda i: tuple(0 for _ in shape))


def _pre_call(x, w_in, b_in, g, b, w0, a0s, a0d):
    grid = (NPAD // BL,)
    return pl.pallas_call(
        _pre_body,
        grid=grid,
        in_specs=[
            _row_spec(IN),
            _full_spec((IN, HID)),
            _full_spec((1, HID)),
            _full_spec((1, HID)),
            _full_spec((1, HID)),
            _full_spec((HID, HID)),
            _full_spec((HID, 16)),
            _full_spec((HID, 16)),
        ],
        out_specs=[_row_spec(HID), _row_spec(HID), _row_spec(16),
                   _row_spec(16)],
        out_shape=[
            jax.ShapeDtypeStruct((NPAD, HID), jnp.float32),
            jax.ShapeDtypeStruct((NPAD, HID), jnp.float32),
            jax.ShapeDtypeStruct((NPAD, 16), jnp.float32),
            jax.ShapeDtypeStruct((NPAD, 16), jnp.float32),
        ],
    )(x, w_in, b_in, g, b, w0, a0s, a0d)


def _mid_call(acc, den, bias, g, b, rmat, hprev, wn, ans, and_, hid_next):
    grid = (NPAD // BL,)
    return pl.pallas_call(
        _mid_body,
        grid=grid,
        in_specs=[
            pl.BlockSpec((2, BL, HID), lambda i: (0, i, 0)),
            pl.BlockSpec((2, BL, 16), lambda i: (0, i, 0)),
            _full_spec((1, HID)),
            _full_spec((1, HID)),
            _full_spec((1, HID)),
            _full_spec((16, HID)),
            _row_spec(HID),
            _full_spec((HID, hid_next)),
            _full_spec((hid_next, 16)),
            _full_spec((hid_next, 16)),
        ],
        out_specs=[_row_spec(HID), _row_spec(hid_next), _row_spec(16),
                   _row_spec(16)],
        out_shape=[
            jax.ShapeDtypeStruct((NPAD, HID), jnp.float32),
            jax.ShapeDtypeStruct((NPAD, hid_next), jnp.float32),
            jax.ShapeDtypeStruct((NPAD, 16), jnp.float32),
            jax.ShapeDtypeStruct((NPAD, 16), jnp.float32),
        ],
    )(acc, den, bias, g, b, rmat, hprev, wn, ans, and_)


def _fin_call(acc, den, bias, g, b, rmat):
    grid = (NPAD // BL,)
    return pl.pallas_call(
        _fin_body,
        grid=grid,
        in_specs=[
            pl.BlockSpec((2, BL, OUT), lambda i: (0, i, 0)),
            pl.BlockSpec((2, BL, 16), lambda i: (0, i, 0)),
            _full_spec((1, OUT)),
            _full_spec((1, OUT)),
            _full_spec((1, OUT)),
            _full_spec((16, OUT)),
        ],
        out_specs=[_row_spec(OUT)],
        out_shape=[jax.ShapeDtypeStruct((NPAD, OUT), jnp.float32)],
    )(acc, den, bias, g, b, rmat)[0]


def _make_edge_pass(hid, ew):
    nreg = hid // 16
    nch = ew // CH
    assert nch % 4 == 0 and nch >= 4
    t_max = nch // 4
    rows_per = NPAD // NS
    nzc = rows_per // RC
    heads8 = hid == HID
    mesh = plsc.VectorSubcoreMesh(core_axis_name="c", subcore_axis_name="s")

    @functools.partial(
        pl.kernel,
        out_type=[
            jax.ShapeDtypeStruct((NC, NPAD, hid), jnp.float32),
            jax.ShapeDtypeStruct((NC, NPAD, 16), jnp.float32),
        ],
        mesh=mesh,
        scratch_types=(
            [pltpu.VMEM((2, CH), jnp.int32)] * 4 +
            [pltpu.VMEM((CH, hid), jnp.float32)] * 2 +
            [pltpu.VMEM((CH, 16), jnp.float32)] * 6 +
            [
                pltpu.VMEM_SHARED((NPAD, hid), jnp.float32),
                pltpu.VMEM_SHARED((NPAD, 16), jnp.float32),
            ] +
            [pltpu.SemaphoreType.DMA] * 5
        ),
        compiler_params=pltpu.CompilerParams(use_tc_tiling_on_sc=False, disable_bounds_checks=True),
    )
    def edge_pass(sd_hbm, hw_hbm, as_hbm, ad_hbm, acc_out, den_out,
                  sd0, sd1, sd2, sd3, hg_a, hg_b,
                  asg_a, asg_b, adg_a, adg_b, exu_a, exu_b,
                  acc_s, den_s, semi, semg_a, semg_b, sems_a, sems_b):
        cid = lax.axis_index("c")
        sid = lax.axis_index("s")
        wid = sid * NC + cid

        sd4 = [sd0, sd1, sd2, sd3]
        sv4 = [r.at[0] for r in sd4]
        dv4 = [r.at[1] for r in sd4]
        hg2 = [hg_a, hg_b]
        asg2 = [asg_a, asg_b]
        adg2 = [adg_a, adg_b]
        exu2 = [exu_a, exu_b]
        semg2 = [semg_a, semg_b]
        sems2 = [sems_a, sems_b]

        zero16 = jnp.zeros((16,), jnp.float32)

        def zero_row(r, _):
            for j in range(nreg):
                hg_a[r, pl.ds(j * 16, 16)] = zero16
            exu_a[r, :] = zero16
            return 0

        lax.fori_loop(0, CH, zero_row, 0)

        base_r = sid * rows_per

        def zero_spmem(k, _):
            pltpu.async_copy(hg_a.at[pl.ds(0, RC)],
                             acc_s.at[pl.ds(base_r + k * RC, RC)], semi)
            pltpu.async_copy(exu_a.at[pl.ds(0, RC)],
                             den_s.at[pl.ds(base_r + k * RC, RC)], semi)
            return 0

        def zero_wait(k, _):
            pltpu.make_async_copy(
                hg_a.at[pl.ds(0, RC)], acc_s.at[pl.ds(base_r, RC)],
                semi).wait()
            pltpu.make_async_copy(
                exu_a.at[pl.ds(0, RC)], den_s.at[pl.ds(base_r, RC)],
                semi).wait()
            return 0

        lax.fori_loop(0, nzc, zero_spmem, 0)
        lax.fori_loop(0, nzc, zero_wait, 0)
        plsc.subcore_barrier()

        bc_idx = [
            jnp.full((16,), j if heads8 else 0, jnp.int32)
            for j in range(nreg)
        ]

        base_c = wid * (ew // CH)

        def issue_idx(ci, sl):
            pltpu.async_copy(sd_hbm.at[base_c + ci], sd4[sl], semi)

        def wait_idx(sl):
            pltpu.make_async_copy(sd_hbm.at[0], sd4[sl], semi).wait()

        def issue_gath(sl, b):
            pltpu.async_copy(hw_hbm.at[sv4[sl]], hg2[b], semg2[b])
            pltpu.async_copy(as_hbm.at[sv4[sl]], asg2[b], semg2[b])
            pltpu.async_copy(ad_hbm.at[dv4[sl]], adg2[b], semg2[b])

        def wait_gath(sl, b):
            pltpu.make_async_copy(hw_hbm.at[sv4[sl]], hg2[b], semg2[b]).wait()
            pltpu.make_async_copy(
                as_hbm.at[sv4[sl]], asg2[b], semg2[b]).wait()
            pltpu.make_async_copy(
                ad_hbm.at[dv4[sl]], adg2[b], semg2[b]).wait()

        def issue_scat(sl, b):
            pltpu.async_copy(hg2[b], acc_s.at[dv4[sl]], sems2[b], add=True)
            pltpu.async_copy(exu2[b], den_s.at[dv4[sl]], sems2[b], add=True)

        def wait_scat(sl, b):
            pltpu.make_async_copy(
                hg2[b], acc_s.at[dv4[sl]], sems2[b]).wait()
            pltpu.make_async_copy(
                exu2[b], den_s.at[dv4[sl]], sems2[b]).wait()

        def compute(b):
            hg, asg, adg, exu = hg2[b], asg2[b], adg2[b], exu2[b]

            @plsc.parallel_loop(0, CH, 1, unroll=4)
            def edge_body(e):
                x = asg[e, :] + adg[e, :]
                ex = jnp.exp(jnp.maximum(x, 0.2 * x))
                exu[e, :] = ex
                for j in range(nreg):
                    bc = ex.at[bc_idx[j]].get(mode="promise_in_bounds")
                    hg[e, pl.ds(j * 16, 16)] = hg[e, pl.ds(j * 16, 16)] * bc

        issue_idx(jnp.int32(0), 0)
        issue_idx(jnp.int32(1), 1)
        wait_idx(0)
        issue_gath(0, 0)

        def body4(t, _):
            for k in range(4):
                g = 4 * t + k
                b = k % 2
                bn = (k + 1) % 2
                sl = k
                sln = (k + 1) % 4
                slp = (k - 1) % 4
                wait_gath(sl, b)
                if k < 3:
                    wait_idx(sln)
                    if k == 0:
                        @pl.when(t > 0)
                        def _():
                            wait_scat(slp, bn)
                    else:
                        wait_scat(slp, bn)
                    issue_gath(sln, bn)
                else:
                    @pl.when(t < t_max - 1)
                    def _():
                        wait_idx(sln)
                        wait_scat(slp, bn)
                        issue_gath(sln, bn)
                compute(b)
                issue_scat(sl, b)
                if k < 2:
                    issue_idx(g + 2, (k + 2) % 4)
                else:
                    @pl.when(t < t_max - 1)
                    def _():
                        issue_idx(g + 2, (k + 2) % 4)
            return 0

        lax.fori_loop(0, t_max, body4, 0)
        wait_scat(2, 0)
        wait_scat(3, 1)
        plsc.subcore_barrier()

        def copy_out(k, _):
            sl = pl.ds(base_r + k * RC, RC)
            pltpu.async_copy(acc_s.at[sl], acc_out.at[cid, sl], semi)
            pltpu.async_copy(den_s.at[sl], den_out.at[cid, sl], semi)
            return 0

        def copy_wait(k, _):
            sl = pl.ds(base_r + k * RC, RC)
            pltpu.make_async_copy(
                acc_s.at[sl], acc_out.at[cid, sl], semi).wait()
            pltpu.make_async_copy(
                den_s.at[sl], den_out.at[cid, sl], semi).wait()
            return 0

        lax.fori_loop(0, nzc, copy_out, 0)
        lax.fori_loop(0, nzc, copy_wait, 0)

    return edge_pass


def _amats(att_s, att_d, hid_in):
    z8 = jnp.zeros((HID, 8), jnp.float32)
    if hid_in == HID:
        asf = att_s.reshape(HID, 1)
        adf = att_d.reshape(HID, 1)
        amat_s = jnp.concatenate([jnp.asarray(_KRON) * asf, z8], axis=1)
        amat_d = jnp.concatenate([jnp.asarray(_KRON) * adf, z8], axis=1)
        return amat_s, amat_d
    z15 = jnp.zeros((OUT, 15), jnp.float32)
    amat_s = jnp.concatenate([att_s.reshape(OUT, 1), z15], axis=1)
    amat_d = jnp.concatenate([att_d.reshape(OUT, 1), z15], axis=1)
    return amat_s, amat_d


def kernel(x, edge_index, W_in, b_in, ln_in_g, ln_in_b, W0, as0, ad0, bc0,
           ln0_g, ln0_b, W1, as1, ad1, bc1, ln1_g, ln1_b, W2, as2, ad2, bc2,
           ln2_g, ln2_b):
    e0 = edge_index.shape[1]
    etot = e0 + N
    nch = 4 * (-(-etot // (NW * CH * 4)))
    ew = nch * CH
    ep = NW * ew
    npad_extra = NPAD - N

    loops = jnp.arange(N, dtype=jnp.int32)
    padc = ep - etot
    pad_idx = (N + (jnp.arange(padc, dtype=jnp.int32) % npad_extra))
    src_all = jnp.concatenate([edge_index[0].astype(jnp.int32), loops, pad_idx])
    dst_all = jnp.concatenate([edge_index[1].astype(jnp.int32), loops, pad_idx])
    sd_all = jnp.stack([src_all.reshape(NW * nch, CH),
                        dst_all.reshape(NW * nch, CH)], axis=1)

    xp = jnp.concatenate([x, jnp.zeros((npad_extra, IN), jnp.float32)], axis=0)

    r2 = lambda v: v.reshape(1, -1)
    r128 = jnp.asarray(_R128)
    r64 = jnp.asarray(_R64)

    edge128 = _make_edge_pass(HID, ew)
    edge64 = _make_edge_pass(OUT, ew)

    a0s, a0d = _amats(as0, ad0, HID)
    a1s, a1d = _amats(as1, ad1, HID)
    a2s, a2d = _amats(as2, ad2, OUT)

    h0, hw0, ast0, adt0 = _pre_call(xp, W_in, r2(b_in), r2(ln_in_g),
                                    r2(ln_in_b), W0, a0s, a0d)
    acc0, den0 = edge128(sd_all, hw0, ast0, adt0)
    h1, hw1, ast1, adt1 = _mid_call(acc0, den0, r2(bc0), r2(ln0_g),
                                    r2(ln0_b), r128, h0, W1, a1s, a1d, HID)
    acc1, den1 = edge128(sd_all, hw1, ast1, adt1)
    h2, hw2, ast2, adt2 = _mid_call(acc1, den1, r2(bc1), r2(ln1_g),
                                    r2(ln1_b), r128, h1, W2, a2s, a2d, OUT)
    acc2, den2 = edge64(sd_all, hw2, ast2, adt2)
    out = _fin_call(acc2, den2, r2(bc2), r2(ln2_g), r2(ln2_b), r64)
    return out[:N]
